# baseline (speedup 1.0000x reference)
import functools

import jax
import jax.numpy as jnp
import numpy as np
from jax.experimental import pallas as pl
from jax.experimental.pallas import tpu as pltpu

_E_IDS = np.array([0, 100, 200, 300, 400, 500, 600, 700, 800, 900, 1000, 1100,
                   1200, 1300, 1400, 1500, 1600, 1700, 1800, 1900, 2000, 2100,
                   2200, 2300, 2400, 2500, 2600, 2700, 2800, 2900, 3000, 3100,
                   3200, 3300, 3400, 3500, 3600, 3700, 3800, 3900], dtype=np.int64)
_E_ANG = np.array([0, 9, 18, 27, 36, 45, 54, 63, 72, 81, 90, 99, 108, 117, 126,
                   135, 144, 153, 162, 171, 180, 189, 198, 207, 216, 225, 234,
                   243, 252, 261, 270, 279, 288, 297, 306, 315, 324, 333, 342,
                   351], dtype=np.float32)
_I_IDS = np.array([50, 150, 250, 350, 450, 550, 650, 750, 850, 950, 1050, 1150,
                   1250, 1350, 1450, 1550, 1650, 1750, 1850, 1950, 2050, 2150,
                   2250, 2350, 2450, 2550, 2650, 2750, 2850, 2950, 3050, 3150,
                   3250, 3350, 3450, 3550, 3650, 3750, 3850, 3950], dtype=np.int64)
_I_ANG = np.array([4, 13, 22, 31, 40, 49, 58, 67, 76, 85, 94, 103, 112, 121,
                   130, 139, 148, 157, 166, 175, 184, 193, 202, 211, 220, 229,
                   238, 247, 256, 265, 274, 283, 292, 301, 310, 319, 328, 337,
                   346, 355], dtype=np.float32)
_MAIN_ANGLES = np.array([0, 45, 90, 135, 180, 225, 270, 315], dtype=np.float32)

_N = 4096
_ROWS = 8 * 2048


def _build_proj() -> np.ndarray:
    rows = []
    for ids, ang in ((_E_IDS, _E_ANG), (_I_IDS, _I_ANG)):
        diff = np.abs(ang[:, None] - _MAIN_ANGLES[None, :])
        min_idx = np.argmin(diff, axis=1)
        closest = _MAIN_ANGLES[min_idx]
        order = np.argsort(closest, kind="stable")
        sorted_angles = closest[order]
        unique_angles, inv = np.unique(sorted_angles, return_inverse=True)
        nseg = int(unique_angles.shape[0])
        cnt = np.bincount(inv, minlength=nseg).astype(np.float32)
        sorted_ids = ids[order]
        for b in range(nseg):
            w = np.zeros(_N, np.float32)
            w[sorted_ids[inv == b]] += 1.0 / cnt[b]
            w[ids] -= 1.0 / float(ids.shape[0])
            rows.append(w)
    w = np.stack(rows).astype(np.float32)
    if w.shape[0] % 8:
        w = np.concatenate([w, np.zeros((8 - w.shape[0] % 8, _N), np.float32)])
    return w


_W = _build_proj()

_BLK = 256
_GRID = _ROWS // _BLK
_NC = 31 * 128


def _loss_body(x_ref, w_ref, out_ref, acc_ref):
    i = pl.program_id(0)

    @pl.when(i == 0)
    def _init():
        acc_ref[...] = jnp.zeros_like(acc_ref)

    x = x_ref[...]
    acc_ref[...] += x.reshape(_BLK // 8, 8, _NC).sum(axis=0)

    @pl.when(i == _GRID - 1)
    def _fin():
        colsum = acc_ref[...].sum(axis=0, keepdims=True)
        q = (w_ref[...] * colsum).sum(axis=1, keepdims=True)
        scale = 1.0 / (float(_ROWS) * float(_ROWS) * 8.0)
        out_ref[...] = (jnp.sum(q * q, keepdims=True) * scale).reshape(1, 1)


@jax.jit
def kernel(_spikes):
    x = _spikes.reshape(_ROWS, _N)
    out = pl.pallas_call(
        _loss_body,
        grid=(_GRID,),
        in_specs=[
            pl.BlockSpec((_BLK, _NC), lambda i: (i, 0)),
            pl.BlockSpec((_W.shape[0], _NC), lambda i: (0, 0)),
        ],
        out_specs=pl.BlockSpec((1, 1), lambda i: (0, 0)),
        out_shape=jax.ShapeDtypeStruct((1, 1), jnp.float32),
        scratch_shapes=[pltpu.VMEM((8, _NC), jnp.float32)],
    )(x, jnp.asarray(_W[:, :_NC]))
    return out[0, 0]


# speedup vs baseline: 1.0680x; 1.0680x over previous
import functools

import jax
import jax.numpy as jnp
import numpy as np
from jax.experimental import pallas as pl
from jax.experimental.pallas import tpu as pltpu

_E_IDS = np.array([0, 100, 200, 300, 400, 500, 600, 700, 800, 900, 1000, 1100,
                   1200, 1300, 1400, 1500, 1600, 1700, 1800, 1900, 2000, 2100,
                   2200, 2300, 2400, 2500, 2600, 2700, 2800, 2900, 3000, 3100,
                   3200, 3300, 3400, 3500, 3600, 3700, 3800, 3900], dtype=np.int64)
_E_ANG = np.array([0, 9, 18, 27, 36, 45, 54, 63, 72, 81, 90, 99, 108, 117, 126,
                   135, 144, 153, 162, 171, 180, 189, 198, 207, 216, 225, 234,
                   243, 252, 261, 270, 279, 288, 297, 306, 315, 324, 333, 342,
                   351], dtype=np.float32)
_I_IDS = np.array([50, 150, 250, 350, 450, 550, 650, 750, 850, 950, 1050, 1150,
                   1250, 1350, 1450, 1550, 1650, 1750, 1850, 1950, 2050, 2150,
                   2250, 2350, 2450, 2550, 2650, 2750, 2850, 2950, 3050, 3150,
                   3250, 3350, 3450, 3550, 3650, 3750, 3850, 3950], dtype=np.int64)
_I_ANG = np.array([4, 13, 22, 31, 40, 49, 58, 67, 76, 85, 94, 103, 112, 121,
                   130, 139, 148, 157, 166, 175, 184, 193, 202, 211, 220, 229,
                   238, 247, 256, 265, 274, 283, 292, 301, 310, 319, 328, 337,
                   346, 355], dtype=np.float32)
_MAIN_ANGLES = np.array([0, 45, 90, 135, 180, 225, 270, 315], dtype=np.float32)

_N = 4096
_ROWS = 8 * 2048


def _build_proj() -> np.ndarray:
    rows = []
    for ids, ang in ((_E_IDS, _E_ANG), (_I_IDS, _I_ANG)):
        diff = np.abs(ang[:, None] - _MAIN_ANGLES[None, :])
        min_idx = np.argmin(diff, axis=1)
        closest = _MAIN_ANGLES[min_idx]
        order = np.argsort(closest, kind="stable")
        sorted_angles = closest[order]
        unique_angles, inv = np.unique(sorted_angles, return_inverse=True)
        nseg = int(unique_angles.shape[0])
        cnt = np.bincount(inv, minlength=nseg).astype(np.float32)
        sorted_ids = ids[order]
        for b in range(nseg):
            w = np.zeros(_N, np.float32)
            w[sorted_ids[inv == b]] += 1.0 / cnt[b]
            w[ids] -= 1.0 / float(ids.shape[0])
            rows.append(w)
    w = np.stack(rows).astype(np.float32)
    if w.shape[0] % 8:
        w = np.concatenate([w, np.zeros((8 - w.shape[0] % 8, _N), np.float32)])
    return w


_W = _build_proj()

_BLK = 512
_GRID = _ROWS // _BLK
_NC = 31 * 128


def _loss_body(x_ref, w_ref, out_ref, acc_ref):
    i = pl.program_id(0)

    @pl.when(i == 0)
    def _init():
        acc_ref[...] = jnp.zeros_like(acc_ref)

    x = x_ref[...]
    acc_ref[...] += x.reshape(_BLK // 8, 8, _NC).sum(axis=0)

    @pl.when(i == _GRID - 1)
    def _fin():
        colsum = acc_ref[...].sum(axis=0, keepdims=True)
        q = (w_ref[...] * colsum).sum(axis=1, keepdims=True)
        scale = 1.0 / (float(_ROWS) * float(_ROWS) * 8.0)
        out_ref[...] = (jnp.sum(q * q, keepdims=True) * scale).reshape(1, 1)


@jax.jit
def kernel(_spikes):
    x = _spikes.reshape(_ROWS, _N)
    out = pl.pallas_call(
        _loss_body,
        grid=(_GRID,),
        in_specs=[
            pl.BlockSpec((_BLK, _NC), lambda i: (i, 0)),
            pl.BlockSpec((_W.shape[0], _NC), lambda i: (0, 0)),
        ],
        out_specs=pl.BlockSpec((1, 1), lambda i: (0, 0)),
        out_shape=jax.ShapeDtypeStruct((1, 1), jnp.float32),
        scratch_shapes=[pltpu.VMEM((8, _NC), jnp.float32)],
    )(x, jnp.asarray(_W[:, :_NC]))
    return out[0, 0]
